# baseline (speedup 1.0000x reference)
import functools

import jax
import jax.numpy as jnp
from jax import lax
from jax.experimental import pallas as pl
from jax.experimental.pallas import tpu as pltpu
from jax.experimental.pallas import tpu_sc as plsc

B = 16
N = 100000
C = 3
UP = 1
EPS = 0.01

NC = 2
NS = 16
L = 16

NH = N // 2
CHUNK = 2000
RAW = CHUNK * C
NCHUNK = NH // CHUNK
GROUPS = CHUNK // L


def _body(src_hbm, def_hbm, out_hbm,
          stage_s, stage_d, src_up, def_up,
          shared_min, shared_acc, shared_cnt, vec_buf, grid_buf):
    c = lax.axis_index("c")
    s = lax.axis_index("s")
    row = 8 * c + lax.div(s, 2)
    half = lax.rem(s, 2)
    base = row * (N * C) + half * (NH * C)

    lane = lax.iota(jnp.int32, L)
    gather_base = lane * C + UP

    vmin = jnp.full((L,), jnp.inf, jnp.float32)
    for ci in range(NCHUNK):
        off = base + ci * RAW
        pltpu.sync_copy(src_hbm.at[pl.ds(off, RAW)], stage_s)
        pltpu.sync_copy(def_hbm.at[pl.ds(off, RAW)], stage_d)

        def extract(j, vm, ci=ci):
            idx = gather_base + j * (L * C)
            sv = plsc.load_gather(stage_s, [idx])
            dv = plsc.load_gather(stage_d, [idx])
            o = ci * CHUNK + j * L
            src_up[pl.ds(o, L)] = sv
            def_up[pl.ds(o, L)] = dv
            return jnp.minimum(vm, sv)

        vmin = lax.fori_loop(0, GROUPS, extract, vmin)

    vec_buf[...] = vmin
    pltpu.sync_copy(vec_buf, shared_min.at[pl.ds(s * L, L)])
    plsc.subcore_barrier()
    pltpu.sync_copy(shared_min.at[pl.ds((s ^ 1) * L, L)], vec_buf)
    ground = jnp.min(jnp.minimum(vmin, vec_buf[...]))

    def masked(j, carry):
        acc, cnt = carry
        o = j * L
        sv = src_up[pl.ds(o, L)]
        dv = def_up[pl.ds(o, L)]
        m = jnp.abs(sv - ground) < EPS
        acc = acc + jnp.where(m, jnp.abs(sv - dv), 0.0)
        cnt = cnt + jnp.where(m, 1.0, 0.0)
        return acc, cnt

    zeros = jnp.zeros((L,), jnp.float32)
    acc, cnt = lax.fori_loop(0, NH // L, masked, (zeros, zeros))

    vec_buf[...] = acc
    pltpu.sync_copy(vec_buf, shared_acc.at[pl.ds(s * L, L)])
    vec_buf[...] = cnt
    pltpu.sync_copy(vec_buf, shared_cnt.at[pl.ds(s * L, L)])
    plsc.subcore_barrier()

    @pl.when(s == 0)
    def _():
        ta = jnp.zeros((L,), jnp.float32)
        tc = jnp.zeros((L,), jnp.float32)
        pltpu.sync_copy(shared_acc, grid_buf)
        for r in range(NS):
            ta = ta + grid_buf[pl.ds(r * L, L)]
        pltpu.sync_copy(shared_cnt, grid_buf)
        for r in range(NS):
            tc = tc + grid_buf[pl.ds(r * L, L)]
        tsum = jnp.sum(ta)
        tcnt = jnp.sum(tc)
        vec_buf[...] = jnp.where(lane == 0, tsum, jnp.where(lane == 1, tcnt, 0.0))
        pltpu.sync_copy(vec_buf, out_hbm.at[c])


@functools.partial(
    pl.kernel,
    out_type=jax.ShapeDtypeStruct((NC, L), jnp.float32),
    mesh=plsc.VectorSubcoreMesh(core_axis_name="c", subcore_axis_name="s",
                                num_cores=NC, num_subcores=NS),
    scratch_types=[
        pltpu.VMEM((RAW,), jnp.float32),
        pltpu.VMEM((RAW,), jnp.float32),
        pltpu.VMEM((NH,), jnp.float32),
        pltpu.VMEM((NH,), jnp.float32),
        pltpu.VMEM_SHARED((NS * L,), jnp.float32),
        pltpu.VMEM_SHARED((NS * L,), jnp.float32),
        pltpu.VMEM_SHARED((NS * L,), jnp.float32),
        pltpu.VMEM((L,), jnp.float32),
        pltpu.VMEM((NS * L,), jnp.float32),
    ],
    compiler_params=pltpu.CompilerParams(needs_layout_passes=False),
)
def _grounding_sc(src_hbm, def_hbm, out_hbm, *scratch):
    _body(src_hbm, def_hbm, out_hbm, *scratch)


def kernel(source, deformed):
    partials = _grounding_sc(source.reshape(-1), deformed.reshape(-1))
    return (partials[0, 0] + partials[1, 0]) / (partials[0, 1] + partials[1, 1])


# speedup vs baseline: 260.7851x; 260.7851x over previous
import functools

import jax
import jax.numpy as jnp
from jax import lax
from jax.experimental import pallas as pl
from jax.experimental.pallas import tpu as pltpu
from jax.experimental.pallas import tpu_sc as plsc

B = 16
N = 100000
UP = 1
EPS = 0.01

NC = 2
NS = 16
L = 16
R = B // NC

W = 6272
WLAST = 5888
NV = W // L
NVLAST = WLAST // L
TAIL0 = (N // 128) * 128
TAIL = N - TAIL0
NT = TAIL // L


def _body(src_hbm, def_hbm, tails_hbm, taild_hbm, out_hbm,
          cache_s, cache_d, tail_cs, tail_cd, min_buf, grid_buf,
          shared_min, shared_acc, shared_cnt, vec_buf):
    c = lax.axis_index("c")
    s = lax.axis_index("s")
    band = c
    col0 = s * W
    lane = lax.iota(jnp.int32, L)

    @pl.when(s < NS - 1)
    def _():
        pltpu.sync_copy(src_hbm.at[UP, pl.ds(band * R, R), pl.ds(col0, W)],
                        cache_s.at[:, pl.ds(0, W)])
        pltpu.sync_copy(def_hbm.at[UP, pl.ds(band * R, R), pl.ds(col0, W)],
                        cache_d.at[:, pl.ds(0, W)])

    @pl.when(s == NS - 1)
    def _():
        pltpu.sync_copy(src_hbm.at[UP, pl.ds(band * R, R), pl.ds(col0, WLAST)],
                        cache_s.at[:, pl.ds(0, WLAST)])
        pltpu.sync_copy(def_hbm.at[UP, pl.ds(band * R, R), pl.ds(col0, WLAST)],
                        cache_d.at[:, pl.ds(0, WLAST)])
        pltpu.sync_copy(tails_hbm.at[pl.ds(band * R, R), :], tail_cs)
        pltpu.sync_copy(taild_hbm.at[pl.ds(band * R, R), :], tail_cd)

    nv = jnp.where(s == NS - 1, NVLAST, NV)

    def phase1(j, vm):
        o = j * L
        return tuple(jnp.minimum(vm[r], cache_s[r, pl.ds(o, L)])
                     for r in range(R))

    inf = jnp.full((L,), jnp.inf, jnp.float32)
    vmin = list(lax.fori_loop(0, nv, phase1, (inf,) * R))

    is_last = s == NS - 1
    for r in range(R):
        for g in range(NT):
            tv = tail_cs[r, pl.ds(g * L, L)]
            vmin[r] = jnp.where(is_last, jnp.minimum(vmin[r], tv), vmin[r])

    for r in range(R):
        min_buf[pl.ds(r * L, L)] = vmin[r]
    pltpu.sync_copy(min_buf, shared_min.at[pl.ds(s * (R * L), R * L)])
    plsc.subcore_barrier()
    pltpu.sync_copy(shared_min, grid_buf)
    ground = []
    for r in range(R):
        red = inf
        for w in range(NS):
            red = jnp.minimum(red, grid_buf[pl.ds(w * (R * L) + r * L, L)])
        ground.append(jnp.min(red))

    def phase2(j, carry):
        acc, cnt = carry
        o = j * L
        new_acc, new_cnt = [], []
        for r in range(R):
            sv = cache_s[r, pl.ds(o, L)]
            dv = cache_d[r, pl.ds(o, L)]
            m = jnp.abs(sv - ground[r]) < EPS
            new_acc.append(acc[r] + jnp.where(m, jnp.abs(sv - dv), 0.0))
            new_cnt.append(cnt[r] + jnp.where(m, 1.0, 0.0))
        return tuple(new_acc), tuple(new_cnt)

    zero = jnp.zeros((L,), jnp.float32)
    acc, cnt = lax.fori_loop(0, nv, phase2, ((zero,) * R, (zero,) * R))
    tot_acc = zero
    tot_cnt = zero
    for r in range(R):
        tot_acc = tot_acc + acc[r]
        tot_cnt = tot_cnt + cnt[r]

    for r in range(R):
        for g in range(NT):
            sv = tail_cs[r, pl.ds(g * L, L)]
            dv = tail_cd[r, pl.ds(g * L, L)]
            m = (jnp.abs(sv - ground[r]) < EPS) & is_last
            tot_acc = tot_acc + jnp.where(m, jnp.abs(sv - dv), 0.0)
            tot_cnt = tot_cnt + jnp.where(m, 1.0, 0.0)

    vec_buf[...] = tot_acc
    pltpu.sync_copy(vec_buf, shared_acc.at[pl.ds(s * L, L)])
    vec_buf[...] = tot_cnt
    pltpu.sync_copy(vec_buf, shared_cnt.at[pl.ds(s * L, L)])
    plsc.subcore_barrier()

    @pl.when(s == 0)
    def _():
        ta = jnp.zeros((L,), jnp.float32)
        tc = jnp.zeros((L,), jnp.float32)
        pltpu.sync_copy(shared_acc, grid_buf.at[pl.ds(0, NS * L)])
        for w in range(NS):
            ta = ta + grid_buf[pl.ds(w * L, L)]
        pltpu.sync_copy(shared_cnt, grid_buf.at[pl.ds(0, NS * L)])
        for w in range(NS):
            tc = tc + grid_buf[pl.ds(w * L, L)]
        tsum = jnp.sum(ta)
        tcnt = jnp.sum(tc)
        vec_buf[...] = jnp.where(lane == 0, tsum, jnp.where(lane == 1, tcnt, 0.0))
        pltpu.sync_copy(vec_buf, out_hbm.at[c])


@functools.partial(
    pl.kernel,
    out_type=jax.ShapeDtypeStruct((NC, L), jnp.float32),
    mesh=plsc.VectorSubcoreMesh(core_axis_name="c", subcore_axis_name="s",
                                num_cores=NC, num_subcores=NS),
    scratch_types=[
        pltpu.VMEM((R, W), jnp.float32),
        pltpu.VMEM((R, W), jnp.float32),
        pltpu.VMEM((R, TAIL), jnp.float32),
        pltpu.VMEM((R, TAIL), jnp.float32),
        pltpu.VMEM((R * L,), jnp.float32),
        pltpu.VMEM((NS * R * L,), jnp.float32),
        pltpu.VMEM_SHARED((NS * R * L,), jnp.float32),
        pltpu.VMEM_SHARED((NS * L,), jnp.float32),
        pltpu.VMEM_SHARED((NS * L,), jnp.float32),
        pltpu.VMEM((L,), jnp.float32),
    ],
    compiler_params=pltpu.CompilerParams(needs_layout_passes=False,
                                         use_tc_tiling_on_sc=True),
)
def _grounding_sc(src_hbm, def_hbm, tails_hbm, taild_hbm, out_hbm, *scratch):
    _body(src_hbm, def_hbm, tails_hbm, taild_hbm, out_hbm, *scratch)


def kernel(source, deformed):
    st = jnp.transpose(source, (2, 0, 1))
    dt = jnp.transpose(deformed, (2, 0, 1))
    tail_s = source[:, TAIL0:, UP]
    tail_d = deformed[:, TAIL0:, UP]
    partials = _grounding_sc(st, dt, tail_s, tail_d)
    return (partials[0, 0] + partials[1, 0]) / (partials[0, 1] + partials[1, 1])


# speedup vs baseline: 272.2649x; 1.0440x over previous
import functools

import jax
import jax.numpy as jnp
from jax import lax
from jax.experimental import pallas as pl
from jax.experimental.pallas import tpu as pltpu
from jax.experimental.pallas import tpu_sc as plsc

B = 16
N = 100000
UP = 1
EPS = 0.01

NC = 2
NS = 16
L = 16
R = B // NC

W = 6272
WLAST = 5888
CA = 2944
NVA = CA // L
NVB = (W - CA) // L
NVBLAST = (WLAST - CA) // L
TAIL0 = (N // 128) * 128
TAIL = N - TAIL0
NT = TAIL // L


def _body(src_hbm, def_hbm, tail_hbm, out_hbm,
          cache_s, cache_d, tail_c, min_buf, grid_buf,
          shared_min, shared_acc, shared_cnt, vec_buf,
          sem_sa, sem_sb, sem_d):
    c = lax.axis_index("c")
    s = lax.axis_index("s")
    band = c
    col0 = s * W
    lane = lax.iota(jnp.int32, L)
    rows = pl.ds(band * R, R)
    is_last = s == NS - 1

    @pl.when(jnp.logical_not(is_last))
    def _():
        pltpu.async_copy(src_hbm.at[UP, rows, pl.ds(col0, CA)],
                         cache_s.at[:, pl.ds(0, CA)], sem_sa)
        pltpu.async_copy(src_hbm.at[UP, rows, pl.ds(col0 + CA, W - CA)],
                         cache_s.at[:, pl.ds(CA, W - CA)], sem_sb)
        pltpu.async_copy(def_hbm.at[UP, rows, pl.ds(col0, W)],
                         cache_d.at[:, pl.ds(0, W)], sem_d)

    @pl.when(is_last)
    def _():
        pltpu.async_copy(src_hbm.at[UP, rows, pl.ds(col0, CA)],
                         cache_s.at[:, pl.ds(0, CA)], sem_sa)
        pltpu.async_copy(src_hbm.at[UP, rows, pl.ds(col0 + CA, WLAST - CA)],
                         cache_s.at[:, pl.ds(CA, WLAST - CA)], sem_sb)
        pltpu.async_copy(def_hbm.at[UP, rows, pl.ds(col0, WLAST)],
                         cache_d.at[:, pl.ds(0, WLAST)], sem_d)
        pltpu.sync_copy(tail_hbm.at[0, rows, :], tail_c.at[pl.ds(0, R), :])
        pltpu.sync_copy(tail_hbm.at[1, rows, :], tail_c.at[pl.ds(R, R), :])

    inf = jnp.full((L,), jnp.inf, jnp.float32)

    def phase1(j, vm):
        o = j * L
        return tuple(jnp.minimum(vm[r], cache_s[r, pl.ds(o, L)])
                     for r in range(R))

    pltpu.make_async_copy(src_hbm.at[UP, rows, pl.ds(col0, CA)],
                          cache_s.at[:, pl.ds(0, CA)], sem_sa).wait()
    vmin = lax.fori_loop(0, NVA, phase1, (inf,) * R)

    @pl.when(jnp.logical_not(is_last))
    def _():
        pltpu.make_async_copy(src_hbm.at[UP, rows, pl.ds(col0 + CA, W - CA)],
                              cache_s.at[:, pl.ds(CA, W - CA)], sem_sb).wait()

    @pl.when(is_last)
    def _():
        pltpu.make_async_copy(src_hbm.at[UP, rows, pl.ds(col0 + CA, WLAST - CA)],
                              cache_s.at[:, pl.ds(CA, WLAST - CA)], sem_sb).wait()

    def phase1b(j, vm):
        o = CA + j * L
        return tuple(jnp.minimum(vm[r], cache_s[r, pl.ds(o, L)])
                     for r in range(R))

    nvb = jnp.where(is_last, NVBLAST, NVB)
    vmin = list(lax.fori_loop(0, nvb, phase1b, tuple(vmin)))

    for r in range(R):
        for g in range(NT):
            tv = tail_c[r, pl.ds(g * L, L)]
            vmin[r] = jnp.where(is_last, jnp.minimum(vmin[r], tv), vmin[r])

    for r in range(R):
        min_buf[pl.ds(r * L, L)] = vmin[r]
    pltpu.sync_copy(min_buf, shared_min.at[pl.ds(s * (R * L), R * L)])
    plsc.subcore_barrier()
    pltpu.sync_copy(shared_min, grid_buf)
    ground = []
    for r in range(R):
        red = inf
        for w in range(NS):
            red = jnp.minimum(red, grid_buf[pl.ds(w * (R * L) + r * L, L)])
        ground.append(jnp.min(red))

    @pl.when(jnp.logical_not(is_last))
    def _():
        pltpu.make_async_copy(def_hbm.at[UP, rows, pl.ds(col0, W)],
                              cache_d.at[:, pl.ds(0, W)], sem_d).wait()

    @pl.when(is_last)
    def _():
        pltpu.make_async_copy(def_hbm.at[UP, rows, pl.ds(col0, WLAST)],
                              cache_d.at[:, pl.ds(0, WLAST)], sem_d).wait()

    def phase2(j, carry):
        acc, cnt = carry
        o = j * L
        new_acc, new_cnt = [], []
        for r in range(R):
            sv = cache_s[r, pl.ds(o, L)]
            dv = cache_d[r, pl.ds(o, L)]
            m = jnp.abs(sv - ground[r]) < EPS
            new_acc.append(acc[r] + jnp.where(m, jnp.abs(sv - dv), 0.0))
            new_cnt.append(cnt[r] + jnp.where(m, 1.0, 0.0))
        return tuple(new_acc), tuple(new_cnt)

    nv = jnp.where(is_last, WLAST // L, W // L)
    zero = jnp.zeros((L,), jnp.float32)
    acc, cnt = lax.fori_loop(0, nv, phase2, ((zero,) * R, (zero,) * R))
    tot_acc = zero
    tot_cnt = zero
    for r in range(R):
        tot_acc = tot_acc + acc[r]
        tot_cnt = tot_cnt + cnt[r]

    for r in range(R):
        for g in range(NT):
            sv = tail_c[r, pl.ds(g * L, L)]
            dv = tail_c[R + r, pl.ds(g * L, L)]
            m = (jnp.abs(sv - ground[r]) < EPS) & is_last
            tot_acc = tot_acc + jnp.where(m, jnp.abs(sv - dv), 0.0)
            tot_cnt = tot_cnt + jnp.where(m, 1.0, 0.0)

    vec_buf[...] = tot_acc
    pltpu.sync_copy(vec_buf, shared_acc.at[pl.ds(s * L, L)])
    vec_buf[...] = tot_cnt
    pltpu.sync_copy(vec_buf, shared_cnt.at[pl.ds(s * L, L)])
    plsc.subcore_barrier()

    @pl.when(s == 0)
    def _():
        ta = jnp.zeros((L,), jnp.float32)
        tc = jnp.zeros((L,), jnp.float32)
        pltpu.sync_copy(shared_acc, grid_buf.at[pl.ds(0, NS * L)])
        for w in range(NS):
            ta = ta + grid_buf[pl.ds(w * L, L)]
        pltpu.sync_copy(shared_cnt, grid_buf.at[pl.ds(0, NS * L)])
        for w in range(NS):
            tc = tc + grid_buf[pl.ds(w * L, L)]
        tsum = jnp.sum(ta)
        tcnt = jnp.sum(tc)
        vec_buf[...] = jnp.where(lane == 0, tsum, jnp.where(lane == 1, tcnt, 0.0))
        pltpu.sync_copy(vec_buf, out_hbm.at[c])


@functools.partial(
    pl.kernel,
    out_type=jax.ShapeDtypeStruct((NC, L), jnp.float32),
    mesh=plsc.VectorSubcoreMesh(core_axis_name="c", subcore_axis_name="s",
                                num_cores=NC, num_subcores=NS),
    scratch_types=[
        pltpu.VMEM((R, W), jnp.float32),
        pltpu.VMEM((R, W), jnp.float32),
        pltpu.VMEM((2 * R, TAIL), jnp.float32),
        pltpu.VMEM((R * L,), jnp.float32),
        pltpu.VMEM((NS * R * L,), jnp.float32),
        pltpu.VMEM_SHARED((NS * R * L,), jnp.float32),
        pltpu.VMEM_SHARED((NS * L,), jnp.float32),
        pltpu.VMEM_SHARED((NS * L,), jnp.float32),
        pltpu.VMEM((L,), jnp.float32),
        pltpu.SemaphoreType.DMA,
        pltpu.SemaphoreType.DMA,
        pltpu.SemaphoreType.DMA,
    ],
    compiler_params=pltpu.CompilerParams(needs_layout_passes=False,
                                         use_tc_tiling_on_sc=True),
)
def _grounding_sc(src_hbm, def_hbm, tail_hbm, out_hbm, *scratch):
    _body(src_hbm, def_hbm, tail_hbm, out_hbm, *scratch)


def kernel(source, deformed):
    st = jnp.transpose(source, (2, 0, 1))
    dt = jnp.transpose(deformed, (2, 0, 1))
    tails = jnp.stack([source[:, TAIL0:, UP], deformed[:, TAIL0:, UP]])
    partials = _grounding_sc(st, dt, tails)
    return (partials[0, 0] + partials[1, 0]) / (partials[0, 1] + partials[1, 1])
